# baseline (speedup 1.0000x reference)
import functools

import jax
import jax.numpy as jnp
from jax import lax
from jax.experimental import pallas as pl
from jax.experimental.pallas import tpu as pltpu
from jax.experimental.pallas import tpu_sc as plsc


@functools.lru_cache(maxsize=None)
def _make_gather(vocab: int, embed_dim: int, batch: int):
    info = plsc.get_sparse_core_info()
    mesh = plsc.ScalarSubcoreMesh(axis_name="c", num_cores=info.num_cores)
    half = batch // info.num_cores

    @functools.partial(
        pl.kernel,
        mesh=mesh,
        out_type=jax.ShapeDtypeStruct((batch, embed_dim), jnp.float32),
    )
    def gather_kernel(idx_hbm, table_hbm, out_hbm):
        cid = lax.axis_index("c")
        base = cid * half
        pltpu.sync_copy(
            table_hbm.at[pl.ds(base, half)], out_hbm.at[pl.ds(base, half)]
        )

    return gather_kernel


def kernel(indices, kernel):
    table = kernel
    vocab, embed_dim = table.shape
    (batch,) = indices.shape
    gather_fn = _make_gather(vocab, embed_dim, batch)
    idx = jnp.asarray(indices, jnp.int32)
    return gather_fn(idx, table)


# speedup vs baseline: 1.3262x; 1.3262x over previous
import functools

import jax
import jax.numpy as jnp
from jax import lax
from jax.experimental import pallas as pl
from jax.experimental.pallas import tpu as pltpu


@functools.lru_cache(maxsize=None)
def _make_gather(vocab: int, embed_dim: int, batch: int):
    half = batch // 2

    def body(idx_ref, table_ref, out_ref, vbuf, sem_v, sem_h, sem_b):
        def loop(i, carry):
            rv = idx_ref[i]
            pltpu.make_async_copy(
                table_ref.at[pl.ds(rv, 1)], vbuf.at[pl.ds(i, 1)], sem_v
            ).start()
            rh = idx_ref[half + i]
            pltpu.make_async_copy(
                table_ref.at[pl.ds(rh, 1)],
                out_ref.at[pl.ds(half + i, 1)],
                sem_h,
            ).start()
            return carry

        lax.fori_loop(0, half, loop, 0, unroll=8)
        pltpu.make_async_copy(table_ref.at[pl.ds(0, half)], vbuf, sem_v).wait()
        pltpu.make_async_copy(
            table_ref.at[pl.ds(0, half)], out_ref.at[pl.ds(half, half)], sem_h
        ).wait()
        copy_b = pltpu.make_async_copy(
            vbuf, out_ref.at[pl.ds(0, half)], sem_b
        )
        copy_b.start()
        copy_b.wait()

    grid_spec = pltpu.PrefetchScalarGridSpec(
        num_scalar_prefetch=1,
        grid=(1,),
        in_specs=[pl.BlockSpec(memory_space=pl.ANY)],
        out_specs=pl.BlockSpec(memory_space=pl.ANY),
        scratch_shapes=[
            pltpu.VMEM((half, embed_dim), jnp.float32),
            pltpu.SemaphoreType.DMA,
            pltpu.SemaphoreType.DMA,
            pltpu.SemaphoreType.DMA,
        ],
    )
    return pl.pallas_call(
        body,
        grid_spec=grid_spec,
        out_shape=jax.ShapeDtypeStruct((batch, embed_dim), jnp.float32),
    )


def kernel(indices, kernel):
    table = kernel
    vocab, embed_dim = table.shape
    (batch,) = indices.shape
    gather_fn = _make_gather(vocab, embed_dim, batch)
    idx = jnp.asarray(indices, jnp.int32)
    return gather_fn(idx, table)


# speedup vs baseline: 1.7836x; 1.3448x over previous
import functools

import jax
import jax.numpy as jnp
from jax import lax
from jax.experimental import pallas as pl
from jax.experimental.pallas import tpu as pltpu
from jax.experimental.pallas import tpu_sc as plsc


@functools.lru_cache(maxsize=None)
def _make_gather(vocab: int, embed_dim: int, batch: int):
    info = plsc.get_sparse_core_info()
    num_workers = info.num_cores * info.num_subcores
    assert batch % (num_workers * 16) == 0
    b_per_w = batch // num_workers

    mesh = plsc.VectorSubcoreMesh(core_axis_name="c", subcore_axis_name="s")

    @functools.partial(
        pl.kernel,
        mesh=mesh,
        out_type=jax.ShapeDtypeStruct((batch, embed_dim), jnp.float32),
        scratch_types=[
            pltpu.VMEM((b_per_w,), jnp.int32),
            pltpu.VMEM((b_per_w, embed_dim), jnp.float32),
            pltpu.SemaphoreType.DMA,
        ],
    )
    def gather_kernel(idx_hbm, table_hbm, out_hbm, idx_v, rows_v, sem):
        wid = lax.axis_index("s") * info.num_cores + lax.axis_index("c")
        base = wid * b_per_w
        pltpu.sync_copy(idx_hbm.at[pl.ds(base, b_per_w)], idx_v)

        def body(k, carry):
            v = idx_v[pl.ds(k * 16, 16)]
            for l in range(16):
                r = v[l]
                pltpu.async_copy(
                    table_hbm.at[pl.ds(r, 1)],
                    rows_v.at[pl.ds(k * 16 + l, 1)],
                    sem,
                )
            return carry

        lax.fori_loop(0, b_per_w // 16, body, 0)
        pltpu.make_async_copy(
            table_hbm.at[pl.ds(0, b_per_w)], rows_v, sem
        ).wait()
        pltpu.sync_copy(rows_v, out_hbm.at[pl.ds(base, b_per_w)])

    return gather_kernel


def kernel(indices, kernel):
    table = kernel
    vocab, embed_dim = table.shape
    (batch,) = indices.shape
    gather_kernel = _make_gather(vocab, embed_dim, batch)
    idx = jnp.asarray(indices, jnp.int32)
    return gather_kernel(idx, table)
